# baseline (speedup 1.0000x reference)
import functools

import jax
import jax.numpy as jnp
from jax import lax
from jax.experimental import pallas as pl
from jax.experimental.pallas import tpu as pltpu
from jax.experimental.pallas import tpu_sc as plsc

VOCAB = 100000
EMB = 64
CTX = 200
LANES = 16

BLK = 4096
NEG = EMB // LANES


def _gather_body(idx2_hbm, par_hbm, wproj_hbm, out_hbm,
                 idx_v, par_v, rows_v, emb_v, sem):
    nc = lax.axis_size("c")
    wid = lax.axis_index("s") * nc + lax.axis_index("c")

    @pl.when(wid == 0)
    def _():
        pltpu.sync_copy(idx2_hbm, idx_v)
        pltpu.sync_copy(par_hbm, par_v)
        pltpu.async_copy(wproj_hbm.at[idx_v.at[0]],
                         rows_v.at[pl.ds(0, 100)], sem).wait()
        pltpu.async_copy(wproj_hbm.at[idx_v.at[1]],
                         rows_v.at[pl.ds(100, 100)], sem).wait()

        zero = jnp.zeros((LANES,), jnp.float32)

        def accum_row(r, parf, accs):
            out = []
            for g in range(NEG):
                lo = rows_v[r, pl.ds(g * LANES, LANES)]
                hi = rows_v[r, pl.ds(EMB + g * LANES, LANES)]
                out.append(accs[g] + (lo + parf * (hi - lo)))
            return tuple(out)

        def group(q, accs):
            pv = par_v[q, :]
            for l in range(LANES):
                parf = jnp.full((LANES,), pv[l], dtype=jnp.float32)
                accs = accum_row(q * LANES + l, parf, accs)
            return accs

        accs = lax.fori_loop(0, CTX // LANES, group, (zero,) * NEG)
        pv = par_v[CTX // LANES, :]
        for l in range(CTX - (CTX // LANES) * LANES):
            parf = jnp.full((LANES,), pv[l], dtype=jnp.float32)
            accs = accum_row((CTX // LANES) * LANES + l, parf, accs)

        for g in range(NEG):
            emb_v[pl.ds(g * LANES, LANES)] = accs[g]
        pltpu.sync_copy(emb_v, out_hbm)


def _sc_gather_sum(idx2, par, W_proj2):
    mesh = plsc.VectorSubcoreMesh(core_axis_name="c", subcore_axis_name="s")
    run = functools.partial(
        pl.kernel,
        out_type=jax.ShapeDtypeStruct((EMB,), jnp.float32),
        mesh=mesh,
        scratch_types=[
            pltpu.VMEM((2, 100), jnp.int32),
            pltpu.VMEM((14, LANES), jnp.float32),
            pltpu.VMEM((CTX, 2 * EMB), jnp.float32),
            pltpu.VMEM((EMB,), jnp.float32),
            pltpu.SemaphoreType.DMA,
        ],
    )(_gather_body)
    return run(idx2, par, W_proj2)


def _matmul_body(emb_ref, w_ref, o_ref):
    o_ref[...] = jnp.dot(emb_ref[...], w_ref[...],
                         preferred_element_type=jnp.float32)


def _tc_project(emb, W_pred):
    grid = (VOCAB + BLK - 1) // BLK
    return pl.pallas_call(
        _matmul_body,
        grid=(grid,),
        in_specs=[
            pl.BlockSpec((1, EMB), lambda i: (0, 0)),
            pl.BlockSpec((EMB, BLK), lambda i: (0, i)),
        ],
        out_specs=pl.BlockSpec((1, BLK), lambda i: (0, i)),
        out_shape=jax.ShapeDtypeStruct((1, VOCAB), jnp.float32),
    )(emb, W_pred)


def kernel(input, W_proj, W_pred):
    idx = input.astype(jnp.int32)
    idx2 = (idx // 2).reshape(2, 100)
    par = jnp.pad((idx % 2).astype(jnp.float32), (0, 24)).reshape(14, LANES)
    W_proj2 = W_proj.reshape(VOCAB // 2, 2 * EMB)
    emb = _sc_gather_sum(idx2, par, W_proj2).reshape(1, EMB)
    return _tc_project(emb, W_pred)


# speedup vs baseline: 1.3558x; 1.3558x over previous
import functools

import jax
import jax.numpy as jnp
from jax import lax
from jax.experimental import pallas as pl
from jax.experimental.pallas import tpu as pltpu
from jax.experimental.pallas import tpu_sc as plsc

VOCAB = 100000
EMB = 64
CTX = 200
LANES = 16

BLK = 4096
NEG = EMB // LANES
NGROUP = CTX // LANES
NTAIL = CTX - NGROUP * LANES
RING = 32


def _gather_body(idxs_hbm, sub_hbm, wproj_hbm, out_hbm,
                 idx_v, sub_v, slabs_v, emb_v, sem):
    nc = lax.axis_size("c")
    wid = lax.axis_index("s") * nc + lax.axis_index("c")

    @pl.when(wid == 0)
    def _():
        pltpu.sync_copy(idxs_hbm, idx_v)
        pltpu.sync_copy(sub_hbm, sub_v)

        slab_ids = []
        subs = []
        for q in range(NGROUP + 1):
            tv = idx_v[q, :]
            sv = sub_v[q, :]
            for l in range(LANES if q < NGROUP else NTAIL):
                slab_ids.append(tv[l])
                subs.append(sv[l])

        def fire(r):
            return pltpu.async_copy(wproj_hbm.at[slab_ids[r]],
                                    slabs_v.at[r % RING], sem)

        copies = [fire(r) for r in range(RING)]
        accs = [jnp.zeros((LANES,), jnp.float32)] * NEG
        for r in range(CTX):
            copies[r % RING].wait()
            s = subs[r]
            accs = [accs[g] + slabs_v[r % RING, s, pl.ds(g * LANES, LANES)]
                    for g in range(NEG)]
            if r + RING < CTX:
                copies[r % RING] = fire(r + RING)

        for g in range(NEG):
            emb_v[pl.ds(g * LANES, LANES)] = accs[g]
        pltpu.sync_copy(emb_v, out_hbm)


def _sc_gather_sum(idxs, sub, W_proj3):
    mesh = plsc.VectorSubcoreMesh(core_axis_name="c", subcore_axis_name="s")
    run = functools.partial(
        pl.kernel,
        out_type=jax.ShapeDtypeStruct((EMB,), jnp.float32),
        mesh=mesh,
        scratch_types=[
            pltpu.VMEM((NGROUP + 1, LANES), jnp.int32),
            pltpu.VMEM((NGROUP + 1, LANES), jnp.int32),
            pltpu.VMEM((RING, 8, EMB), jnp.float32),
            pltpu.VMEM((EMB,), jnp.float32),
            pltpu.SemaphoreType.DMA,
        ],
    )(_gather_body)
    return run(idxs, sub, W_proj3)


def _matmul_body(emb_ref, w_ref, o_ref):
    o_ref[...] = jnp.dot(emb_ref[...], w_ref[...],
                         preferred_element_type=jnp.float32)


def _tc_project(emb, W_pred):
    grid = (VOCAB + BLK - 1) // BLK
    return pl.pallas_call(
        _matmul_body,
        grid=(grid,),
        in_specs=[
            pl.BlockSpec((1, EMB), lambda i: (0, 0)),
            pl.BlockSpec((EMB, BLK), lambda i: (0, i)),
        ],
        out_specs=pl.BlockSpec((1, BLK), lambda i: (0, i)),
        out_shape=jax.ShapeDtypeStruct((1, VOCAB), jnp.float32),
    )(emb, W_pred)


def kernel(input, W_proj, W_pred):
    idx = input.astype(jnp.int32)
    idxs = jnp.pad(idx // 8, (0, LANES - NTAIL)).reshape(NGROUP + 1, LANES)
    sub = jnp.pad(idx % 8, (0, LANES - NTAIL)).reshape(NGROUP + 1, LANES)
    W_proj3 = W_proj.reshape(VOCAB // 8, 8, EMB)
    emb = _sc_gather_sum(idxs, sub, W_proj3).reshape(1, EMB)
    return _tc_project(emb, W_pred)
